# Initial kernel scaffold; baseline (speedup 1.0000x reference)
import jax
import jax.numpy as jnp
from jax.experimental import pallas as pl


def kernel(y_d, dm_s, dm_f, dm_g, edges_s, edges_f, edges_g, params):
    raise NotImplementedError("write your pallas kernel here")


# speedup vs baseline: 55.8956x; 55.8956x over previous
import functools

import jax
import jax.numpy as jnp
from jax.experimental import pallas as pl
from jax.experimental.pallas import tpu as pltpu

N = 4096
FD = 512
E = 262144
H = 2

RBS = 512
CBS = 512
NBS = 512
NEG = -1e30


def _dot_t(a, b):
    return jax.lax.dot_general(a, b, (((0,), (0,)), ((), ())),
                               preferred_element_type=jnp.float32)


def _mm_body(x_ref, w_ref, o_ref):
    o_ref[...] = jnp.dot(x_ref[...], w_ref[...],
                         preferred_element_type=jnp.float32)


def _matmul(x, w):
    n, k = x.shape
    m = w.shape[1]
    return pl.pallas_call(
        _mm_body,
        grid=(n // NBS,),
        in_specs=[pl.BlockSpec((NBS, k), lambda i: (i, 0)),
                  pl.BlockSpec((k, m), lambda i: (0, 0))],
        out_specs=pl.BlockSpec((NBS, m), lambda i: (i, 0)),
        out_shape=jax.ShapeDtypeStruct((n, m), jnp.float32),
    )(x, w)


def _dis_slice(wcs_ref, start, size):
    w = wcs_ref[0, pl.ds(start, size)] + wcs_ref[1, pl.ds(start, size)]
    return jax.lax.rsqrt(w + 1.0)


def _gcn_body(cnt_ref, dm_ref, x_ref, wcs_ref, b_ref, h_ref, sum_ref):
    c = pl.program_id(0)
    r = pl.program_id(1)
    nr = pl.num_programs(1)
    dis_r = _dis_slice(wcs_ref, r * RBS, RBS)
    w_tile = cnt_ref[...] * dm_ref[...] * dis_r[:, None]
    xr = x_ref[pl.ds(r * RBS, RBS), :]
    part = _dot_t(w_tile, xr)

    @pl.when(r == 0)
    def _():
        h_ref[...] = part

    @pl.when(r > 0)
    def _():
        h_ref[...] += part

    @pl.when(r == nr - 1)
    def _():
        dis_c = _dis_slice(wcs_ref, c * CBS, CBS)
        xc = x_ref[pl.ds(c * CBS, CBS), :]
        out = (dis_c[:, None] * h_ref[...]
               + (dis_c * dis_c)[:, None] * xc + b_ref[0, :][None, :])
        out = jnp.maximum(out, 0.0)
        h_ref[...] = out
        s = jnp.sum(out)

        @pl.when(c == 0)
        def _():
            sum_ref[0, 0] = s

        @pl.when(c > 0)
        def _():
            sum_ref[0, 0] += s


def _gcn1(cnt, dm, x, wcs, b):
    return pl.pallas_call(
        _gcn_body,
        grid=(N // CBS, N // RBS),
        in_specs=[
            pl.BlockSpec((RBS, CBS), lambda c, r: (r, c)),
            pl.BlockSpec((RBS, CBS), lambda c, r: (r, c)),
            pl.BlockSpec((N, FD), lambda c, r: (0, 0)),
            pl.BlockSpec((2, N), lambda c, r: (0, 0)),
            pl.BlockSpec((1, FD), lambda c, r: (0, 0)),
        ],
        out_specs=[
            pl.BlockSpec((CBS, FD), lambda c, r: (c, 0)),
            pl.BlockSpec((1, 1), lambda c, r: (0, 0),
                         memory_space=pltpu.SMEM),
        ],
        out_shape=[
            jax.ShapeDtypeStruct((N, FD), jnp.float32),
            jax.ShapeDtypeStruct((1, 1), jnp.float32),
        ],
    )(cnt, dm, x, wcs, b)


def _mm_aux_body(x_ref, w_ref, att_ref, o_ref, aux_ref):
    out = jnp.dot(x_ref[...], w_ref[...], preferred_element_type=jnp.float32)
    o_ref[...] = out
    hg0 = out[:, 0:FD]
    hg1 = out[:, FD:2 * FD]
    aux_ref[0, :] = jnp.sum(hg0 * att_ref[0, :][None, :], axis=1)
    aux_ref[1, :] = jnp.sum(hg1 * att_ref[1, :][None, :], axis=1)
    aux_ref[2, :] = jnp.sum(hg0 * att_ref[2, :][None, :], axis=1)
    aux_ref[3, :] = jnp.sum(hg1 * att_ref[3, :][None, :], axis=1)
    k0 = jnp.sum(att_ref[4, :] * att_ref[6, :])
    k1 = jnp.sum(att_ref[5, :] * att_ref[7, :])
    aux_ref[4, :] = jnp.full((NBS,), k0, jnp.float32)
    aux_ref[5, :] = jnp.full((NBS,), k1, jnp.float32)
    aux_ref[6, :] = jnp.zeros((NBS,), jnp.float32)
    aux_ref[7, :] = jnp.zeros((NBS,), jnp.float32)


def _mm_aux(x, w, att):
    return pl.pallas_call(
        _mm_aux_body,
        grid=(N // NBS,),
        in_specs=[pl.BlockSpec((NBS, FD), lambda i: (i, 0)),
                  pl.BlockSpec((FD, 3 * FD), lambda i: (0, 0)),
                  pl.BlockSpec((8, FD), lambda i: (0, 0))],
        out_specs=[pl.BlockSpec((NBS, 3 * FD), lambda i: (i, 0)),
                   pl.BlockSpec((8, NBS), lambda i: (0, i))],
        out_shape=[jax.ShapeDtypeStruct((N, 3 * FD), jnp.float32),
                   jax.ShapeDtypeStruct((8, N), jnp.float32)],
    )(x, w, att)


def _flash_body(cnt_ref, dm_ref, hgx_ref, aux_ref, wcs_ref, b_ref,
                h2_ref, h3_ref, s2_ref, s3_ref,
                m_ref, den_ref, accg_ref, accn_ref):
    c = pl.program_id(0)
    r = pl.program_id(1)
    nr = pl.num_programs(1)
    dis_r = _dis_slice(wcs_ref, r * RBS, RBS)
    kh = (jnp.sum(aux_ref[4:5, 0:1]), jnp.sum(aux_ref[5:6, 0:1]))
    cnt = cnt_ref[...]
    dmt = dm_ref[...]

    @pl.when(r == 0)
    def _():
        m_ref[...] = jnp.full((2, CBS), NEG, jnp.float32)
        den_ref[...] = jnp.zeros((2, CBS), jnp.float32)
        accg_ref[...] = jnp.zeros((CBS, 2 * FD), jnp.float32)
        accn_ref[...] = jnp.zeros((CBS, FD), jnp.float32)

    hgr = hgx_ref[pl.ds(r * RBS, RBS), :]
    w_tile = cnt * dmt * dis_r[:, None]
    accn_ref[...] += _dot_t(w_tile, hgr[:, 2 * FD:])
    edge = cnt > 0.0
    for hh in range(H):
        asrc_r = aux_ref[hh:hh + 1, pl.ds(r * RBS, RBS)].reshape(RBS)
        adst_c = aux_ref[2 + hh:3 + hh, pl.ds(c * CBS, CBS)].reshape(CBS)
        M = asrc_r[:, None] + adst_c[None, :] + dmt * kh[hh]
        M = jnp.where(M >= 0, M, 0.2 * M)
        Mm = jnp.where(edge, M, NEG)
        mold = m_ref[hh:hh + 1, :].reshape(CBS)
        mnew = jnp.maximum(mold, jnp.max(Mm, axis=0))
        scale = jnp.exp(mold - mnew)
        P = cnt * jnp.exp(Mm - mnew[None, :])
        sl = slice(hh * FD, (hh + 1) * FD)
        den_ref[hh:hh + 1, :] = (den_ref[hh:hh + 1, :] * scale[None, :]
                                 + jnp.sum(P, axis=0)[None, :])
        accg_ref[:, sl] = (accg_ref[:, sl] * scale[:, None]
                           + _dot_t(P, hgr[:, sl]))
        m_ref[hh:hh + 1, :] = mnew[None, :]

    @pl.when(r == nr - 1)
    def _():
        mean_w = jnp.sum(wcs_ref[0, :] + wcs_ref[1, :]) / E
        dis_c = _dis_slice(wcs_ref, c * CBS, CBS)
        hgc = hgx_ref[pl.ds(c * CBS, CBS), :]
        outs = []
        for hh in range(H):
            asrc_c = aux_ref[hh:hh + 1, pl.ds(c * CBS, CBS)].reshape(CBS)
            adst_c = aux_ref[2 + hh:3 + hh, pl.ds(c * CBS, CBS)].reshape(CBS)
            als = asrc_c + adst_c + mean_w * kh[hh]
            als = jnp.where(als >= 0, als, 0.2 * als)
            mold = m_ref[hh:hh + 1, :].reshape(CBS)
            mnew = jnp.maximum(mold, als)
            scale = jnp.exp(mold - mnew)
            pself = jnp.exp(als - mnew)
            den = den_ref[hh:hh + 1, :].reshape(CBS) * scale + pself
            sl = slice(hh * FD, (hh + 1) * FD)
            acc = (accg_ref[:, sl] * scale[:, None]
                   + pself[:, None] * hgc[:, sl])
            outs.append(acc / (den[:, None] + 1e-16))
        h2 = jnp.maximum((outs[0] + outs[1]) * 0.5 + b_ref[0, :][None, :], 0.0)
        h2_ref[...] = h2
        x2c = hgc[:, 2 * FD:]
        h3 = jnp.maximum(dis_c[:, None] * accn_ref[...]
                         + (dis_c * dis_c)[:, None] * x2c
                         + b_ref[1, :][None, :], 0.0)
        h3_ref[...] = h3
        sa = jnp.sum(h2)
        sb = jnp.sum(h3)

        @pl.when(c == 0)
        def _():
            s2_ref[0, 0] = sa
            s3_ref[0, 0] = sb

        @pl.when(c > 0)
        def _():
            s2_ref[0, 0] += sa
            s3_ref[0, 0] += sb


def _flash(cnt, dm, hgx, aux, wcs, b):
    return pl.pallas_call(
        _flash_body,
        grid=(N // CBS, N // RBS),
        in_specs=[
            pl.BlockSpec((RBS, CBS), lambda c, r: (r, c)),
            pl.BlockSpec((RBS, CBS), lambda c, r: (r, c)),
            pl.BlockSpec((N, 3 * FD), lambda c, r: (0, 0)),
            pl.BlockSpec((8, N), lambda c, r: (0, 0)),
            pl.BlockSpec((2, N), lambda c, r: (0, 0)),
            pl.BlockSpec((2, FD), lambda c, r: (0, 0)),
        ],
        out_specs=[
            pl.BlockSpec((CBS, FD), lambda c, r: (c, 0)),
            pl.BlockSpec((CBS, FD), lambda c, r: (c, 0)),
            pl.BlockSpec((1, 1), lambda c, r: (0, 0),
                         memory_space=pltpu.SMEM),
            pl.BlockSpec((1, 1), lambda c, r: (0, 0),
                         memory_space=pltpu.SMEM),
        ],
        out_shape=[
            jax.ShapeDtypeStruct((N, FD), jnp.float32),
            jax.ShapeDtypeStruct((N, FD), jnp.float32),
            jax.ShapeDtypeStruct((1, 1), jnp.float32),
            jax.ShapeDtypeStruct((1, 1), jnp.float32),
        ],
        scratch_shapes=[
            pltpu.VMEM((2, CBS), jnp.float32),
            pltpu.VMEM((2, CBS), jnp.float32),
            pltpu.VMEM((CBS, 2 * FD), jnp.float32),
            pltpu.VMEM((CBS, FD), jnp.float32),
        ],
    )(cnt, dm, hgx, aux, wcs, b)


def _comb_body(m0, m1, m2, m3, m4, m5, m6, m7, m8,
               sums_ref, fc1w_ref, fc1b_ref, fc2w_ref, fc2b_ref, cnn_ref,
               y_ref, coef_ref):
    @pl.when(pl.program_id(0) == 0)
    def _():
        means = sums_ref[0, :] / (N * FD)
        v1 = jnp.sum(fc1w_ref[...] * means[None, :], axis=1) + fc1b_ref[0, :]
        v1 = jnp.maximum(v1, 0.0)
        v2 = jnp.sum(fc2w_ref[...] * v1[None, :], axis=1) + fc2b_ref[0, :]
        v2 = jax.nn.sigmoid(v2)
        coef_ref[0, :] = v2 * cnn_ref[0, :]

    mats = (m0, m1, m2, m3, m4, m5, m6, m7, m8)
    y = jnp.zeros((NBS, FD), jnp.float32) + cnn_ref[1:2, 0:1]
    for i in range(9):
        y = y + coef_ref[0:1, i:i + 1] * mats[i][...]
    y_ref[...] = y


def _combine(mats, sums, fc1w, fc1b, fc2w, fc2b, cnn):
    mat_spec = pl.BlockSpec((NBS, FD), lambda i: (i, 0))
    return pl.pallas_call(
        _comb_body,
        grid=(N // NBS,),
        in_specs=[mat_spec] * 9 + [
            pl.BlockSpec((1, 16), lambda i: (0, 0)),
            pl.BlockSpec((48, 16), lambda i: (0, 0)),
            pl.BlockSpec((1, 48), lambda i: (0, 0)),
            pl.BlockSpec((16, 48), lambda i: (0, 0)),
            pl.BlockSpec((1, 16), lambda i: (0, 0)),
            pl.BlockSpec((2, 16), lambda i: (0, 0)),
        ],
        out_specs=pl.BlockSpec((NBS, FD), lambda i: (i, 0)),
        out_shape=jax.ShapeDtypeStruct((N, FD), jnp.float32),
        scratch_shapes=[pltpu.VMEM((1, 16), jnp.float32)],
    )(*mats, sums, fc1w, fc1b, fc2w, fc2b, cnn)


def _build_counts(edges, dm):
    row = edges[0].astype(jnp.int32)
    col = edges[1].astype(jnp.int32)
    flat = row * N + col
    cnt = jax.ops.segment_sum(jnp.ones((E,), jnp.float32), flat,
                              num_segments=N * N).reshape(N, N)
    wcs = jax.ops.segment_sum(dm.reshape(-1)[flat], col, num_segments=N)
    return cnt, jnp.stack([wcs, jnp.zeros_like(wcs)])


def kernel(y_d, dm_s, dm_f, dm_g, edges_s, edges_f, edges_g, params):
    p = params
    views = (('s', dm_s, edges_s), ('f', dm_f, edges_f), ('g', dm_g, edges_g))

    wcat1 = jnp.concatenate([p[t + '_gcn1_W'].T for t, _, _ in views], axis=1)
    x1all = _matmul(y_d, wcat1)

    mats = {}
    sums = {}
    for i, (t, dm, edges) in enumerate(views):
        cnt, wcs = _build_counts(edges, dm)
        x1 = x1all[:, i * FD:(i + 1) * FD]
        b1 = p[t + '_gcn1_b'].reshape(1, FD)
        h1, s1 = _gcn1(cnt, dm, x1, wcs, b1)
        g = p[t + '_gat']
        wcat = jnp.concatenate([g['lin_W'].T, p[t + '_gcn2_W'].T], axis=1)
        att = jnp.concatenate([
            g['att_src'][0], g['att_dst'][0], g['att_edge'][0],
            g['lin_edge_W'].reshape(H, FD)], axis=0)
        hgx, aux = _mm_aux(h1, wcat, att)
        bcat = jnp.stack([g['bias'], p[t + '_gcn2_b']])
        h2, h3, s2, s3 = _flash(cnt, dm, hgx, aux, wcs, bcat)
        mats[t] = (h1, h2, h3)
        sums[t] = (s1, s2, s3)

    order = [mats['s'][2], mats['s'][1], mats['s'][0],
             mats['f'][2], mats['f'][1], mats['f'][0],
             mats['g'][2], mats['g'][1], mats['g'][0]]
    sorder = [sums['s'][2], sums['s'][1], sums['s'][0],
              sums['f'][2], sums['f'][1], sums['f'][0],
              sums['g'][2], sums['g'][1], sums['g'][0]]
    sums16 = jnp.concatenate(
        [jnp.concatenate(sorder, axis=1),
         jnp.zeros((1, 7), jnp.float32)], axis=1)

    fc1w = jnp.zeros((48, 16), jnp.float32).at[:45, :9].set(p['fc1_W'])
    fc1b = jnp.zeros((1, 48), jnp.float32).at[0, :45].set(p['fc1_b'])
    fc2w = jnp.zeros((16, 48), jnp.float32).at[:9, :45].set(p['fc2_W'])
    fc2b = jnp.zeros((1, 16), jnp.float32).at[0, :9].set(p['fc2_b'])
    cnn = jnp.zeros((2, 16), jnp.float32)
    cnn = cnn.at[0, :9].set(p['cnn_W']).at[1, 0].set(p['cnn_b'][0])

    return _combine(order, sums16, fc1w, fc1b, fc2w, fc2b, cnn)
